# baseline (speedup 1.0000x reference)
import functools

import jax
import jax.numpy as jnp
from jax import lax
from jax.experimental import pallas as pl
from jax.experimental.pallas import tpu as pltpu, tpu_sc as plsc

_EMB = 1000
_BATCH = 16384
_NC = 2
_NS = 16
_NW = _NC * _NS
_JT = _EMB // 8
_IT = _BATCH // 128
_TILES_PER_W = _IT // _NW
_COLS_PER_W = _BATCH // _NW

_mesh = plsc.VectorSubcoreMesh(core_axis_name="c", subcore_axis_name="s")


def _one_hot_body(labels_hbm, out_hbm, lab_v, buf_v, sem):
    wid = lax.axis_index("s") * _NC + lax.axis_index("c")
    col0 = wid * _COLS_PER_W

    pltpu.sync_copy(labels_hbm.at[pl.ds(col0, _COLS_PER_W)], lab_v)

    zeros = jnp.zeros((16,), jnp.float32)
    ones = jnp.ones((16,), jnp.float32)
    zeros_i = jnp.zeros((16,), jnp.int32)
    iota16 = lax.broadcasted_iota(jnp.int32, (16,), 0)

    def _zero_tile(j, _):
        for jr in range(8):
            for jj in range(8):
                buf_v[j, 0, jr, pl.ds(jj * 16, 16)] = zeros
        return _

    lax.fori_loop(0, _JT, _zero_tile, None)

    def _scatter(c, vals):
        for g in range(128 // 16):
            lab16 = lab_v[pl.ds(c * 128 + g * 16, 16)]
            plsc.store_scatter(
                buf_v,
                [lab16 >> 3, zeros_i, lab16 & 7, g * 16 + iota16],
                vals,
            )

    for c in range(_TILES_PER_W):
        _scatter(c, ones)
        pltpu.async_copy(
            buf_v,
            out_hbm.at[:, pl.ds(wid * _TILES_PER_W + c, 1)],
            sem,
        ).wait()
        if c + 1 < _TILES_PER_W:
            _scatter(c, zeros)


_one_hot_sc = functools.partial(
    pl.kernel,
    out_type=jax.ShapeDtypeStruct((_JT, _IT, 8, 128), jnp.float32),
    mesh=_mesh,
    compiler_params=pltpu.CompilerParams(needs_layout_passes=False),
    scratch_types=[
        pltpu.VMEM((_COLS_PER_W,), jnp.int32),
        pltpu.VMEM((_JT, 1, 8, 128), jnp.float32),
        pltpu.SemaphoreType.DMA,
    ],
)(_one_hot_body)


def kernel(labels):
    tiles = _one_hot_sc(labels)
    return tiles.transpose((1, 3, 0, 2)).reshape(_BATCH, _EMB)


# speedup vs baseline: 1.0605x; 1.0605x over previous
import functools

import jax
import jax.numpy as jnp
from jax import lax
from jax.experimental import pallas as pl
from jax.experimental.pallas import tpu as pltpu, tpu_sc as plsc

_EMB = 1000
_BATCH = 16384
_NC = 2
_NS = 16
_NW = _NC * _NS
_JT = _EMB // 8
_IT = _BATCH // 128
_TILES_PER_W = _IT // _NW
_COLS_PER_W = _BATCH // _NW
_JT0 = 63
_JT1 = _JT - _JT0
_LO1 = _JT0 * 8

_mesh = plsc.VectorSubcoreMesh(core_axis_name="c", subcore_axis_name="s")


def _one_hot_body(labels_hbm, out_hbm, lab_v, buf0_v, buf1_v, sem0, sem1):
    wid = lax.axis_index("s") * _NC + lax.axis_index("c")
    col0 = wid * _COLS_PER_W
    i_base = wid * _TILES_PER_W

    pltpu.sync_copy(labels_hbm.at[pl.ds(col0, _COLS_PER_W)], lab_v)

    zeros = jnp.zeros((16,), jnp.float32)
    ones = jnp.ones((16,), jnp.float32)
    zeros_i = jnp.zeros((16,), jnp.int32)
    iota16 = lax.broadcasted_iota(jnp.int32, (16,), 0)

    def _zero_buf(buf, nj):
        def _row(j, carry):
            for jr in range(8):
                for jj in range(8):
                    buf[j, 0, jr, pl.ds(jj * 16, 16)] = zeros
            return carry

        lax.fori_loop(0, nj, _row, None)

    def _scatter(buf, c, lo, hi, vals):
        def _grp(g, carry):
            lab16 = lab_v[pl.ds(c * 128 + g * 16, 16)]
            rel = lab16 - lo
            mask = (lab16 >= lo) & (lab16 < hi)
            plsc.store_scatter(
                buf,
                [rel >> 3, zeros_i, rel & 7, g * 16 + iota16],
                vals,
                mask=mask,
            )
            return carry

        lax.fori_loop(0, 128 // 16, _grp, None)

    def _dma0(c):
        return pltpu.make_async_copy(
            buf0_v, out_hbm.at[pl.ds(0, _JT0), pl.ds(i_base + c, 1)], sem0
        )

    def _dma1(c):
        return pltpu.make_async_copy(
            buf1_v, out_hbm.at[pl.ds(_JT0, _JT1), pl.ds(i_base + c, 1)], sem1
        )

    _zero_buf(buf0_v, _JT0)
    _scatter(buf0_v, 0, 0, _LO1, ones)
    _dma0(0).start()
    _zero_buf(buf1_v, _JT1)
    _scatter(buf1_v, 0, _LO1, _EMB, ones)
    _dma1(0).start()

    def _tile(c, carry):
        _dma0(c).wait()
        _scatter(buf0_v, c - 1, 0, _LO1, zeros)
        _scatter(buf0_v, c, 0, _LO1, ones)
        _dma0(c).start()
        _dma1(c).wait()
        _scatter(buf1_v, c - 1, _LO1, _EMB, zeros)
        _scatter(buf1_v, c, _LO1, _EMB, ones)
        _dma1(c).start()
        return carry

    lax.fori_loop(1, _TILES_PER_W, _tile, None)

    _dma0(0).wait()
    _dma1(0).wait()


_one_hot_sc = functools.partial(
    pl.kernel,
    out_type=jax.ShapeDtypeStruct((_JT, _IT, 8, 128), jnp.float32),
    mesh=_mesh,
    compiler_params=pltpu.CompilerParams(needs_layout_passes=False),
    scratch_types=[
        pltpu.VMEM((_COLS_PER_W,), jnp.int32),
        pltpu.VMEM((_JT0, 1, 8, 128), jnp.float32),
        pltpu.VMEM((_JT1, 1, 8, 128), jnp.float32),
        pltpu.SemaphoreType.DMA,
        pltpu.SemaphoreType.DMA,
    ],
)(_one_hot_body)


def kernel(labels):
    tiles = _one_hot_sc(labels)
    return tiles.transpose((1, 3, 0, 2)).reshape(_BATCH, _EMB)
